# baseline (speedup 1.0000x reference)
import functools

import jax
import jax.numpy as jnp
from jax import lax
from jax.experimental import pallas as pl
from jax.experimental.pallas import tpu as pltpu
from jax.experimental.pallas import tpu_sc as plsc

B_, L_, D_ = 4, 2048, 1024
E_, K_ = 64, 2
T_ = B_ * L_
N_ = T_ * K_

TM = 128
SMAX = N_ // TM + E_
NP_ = SMAX * TM

TG = 512
NT = T_ // TG


def _gate_kernel(x_ref, wg_ref, g0_ref, g1_ref, s0_ref, s1_ref, meta_ref,
                 tril_ref, trilE_ref, i0s, i1s, r0s, r1s, g0s, g1s, cnt_ref):
    s = pl.program_id(0)
    b = jnp.where(s < NT, s, s - NT)

    @pl.when(s == 0)
    def _():
        row = lax.broadcasted_iota(jnp.int32, (TG, TG), 0)
        cc = lax.broadcasted_iota(jnp.int32, (TG, TG), 1)
        tril_ref[...] = (row > cc).astype(jnp.float32)
        er = lax.broadcasted_iota(jnp.int32, (E_, E_), 0)
        ec = lax.broadcasted_iota(jnp.int32, (E_, E_), 1)
        trilE_ref[...] = (er < ec).astype(jnp.float32)

    @pl.when(s < NT)
    def _():
        logits = jnp.dot(x_ref[...], wg_ref[...],
                         preferred_element_type=jnp.float32)
        sig = jax.nn.sigmoid(logits)
        col = lax.broadcasted_iota(jnp.int32, (TG, E_), 1)
        m1 = jnp.max(sig, axis=1, keepdims=True)
        i1 = jnp.min(jnp.where(sig == m1, col, E_), axis=1, keepdims=True)
        sig2 = jnp.where(col == i1, -1.0, sig)
        m2 = jnp.max(sig2, axis=1, keepdims=True)
        i2 = jnp.min(jnp.where(sig2 == m2, col, E_), axis=1, keepdims=True)
        o1 = (col == i1).astype(jnp.float32)
        o2 = (col == i2).astype(jnp.float32)
        o = o1 + o2
        cex = jnp.dot(tril_ref[...], o,
                      preferred_element_type=jnp.float32)
        prev = jnp.where(s == 0, 0.0, cnt_ref[...])
        r1 = jnp.sum((cex + prev) * o1, axis=1, keepdims=True)
        r2 = jnp.sum((cex + prev) * o2, axis=1, keepdims=True)
        i0s[b] = i1
        i1s[b] = i2
        r0s[b] = r1
        r1s[b] = r2
        g0s[b] = m1
        g1s[b] = m2
        cnt_ref[...] = prev + jnp.sum(o, axis=0, keepdims=True)

    @pl.when(s >= NT)
    def _():
        @pl.when(s == NT)
        def _():
            counts = cnt_ref[...]
            tcnt = jnp.floor((counts + (TM - 1)) * (1.0 / TM))
            base = jnp.dot(tcnt, trilE_ref[...],
                           preferred_element_type=jnp.float32)
            stot = jnp.sum(tcnt)
            s2 = lax.broadcasted_iota(jnp.int32, (SMAX, E_), 0).astype(jnp.float32)
            in_e = (s2 >= base) & (s2 < base + tcnt)
            eidsf = lax.broadcasted_iota(jnp.int32, (SMAX, E_), 1).astype(jnp.float32)
            gid = jnp.sum(jnp.where(in_e, eidsf, 0.0), axis=1)
            validv = jnp.sum(in_e.astype(jnp.float32), axis=1)
            sv1 = lax.broadcasted_iota(jnp.int32, (SMAX,), 0).astype(jnp.float32)
            lg = jnp.sum(jnp.where(sv1 == (stot - 1.0), gid, 0.0))
            meta_ref[0, :] = jnp.where(validv > 0, sv1, stot - 1.0).astype(jnp.int32)
            meta_ref[1, :] = jnp.where(validv > 0, gid, lg).astype(jnp.int32)
            meta_ref[2, :] = validv.astype(jnp.int32)
            cnt_ref[...] = base * TM

        col = lax.broadcasted_iota(jnp.int32, (TG, E_), 1)
        opad = cnt_ref[...]
        oh0 = (i0s[b] == col).astype(jnp.float32)
        oh1 = (i1s[b] == col).astype(jnp.float32)
        slot0 = jnp.sum(oh0 * opad, axis=1, keepdims=True) + r0s[b]
        slot1 = jnp.sum(oh1 * opad, axis=1, keepdims=True) + r1s[b]
        s0_ref[...] = jnp.reshape(slot0, (TG,)).astype(jnp.int32)
        s1_ref[...] = jnp.reshape(slot1, (TG,)).astype(jnp.int32)
        g0_ref[...] = jnp.reshape(g0s[b], (TG,))
        g1_ref[...] = jnp.reshape(g1s[b], (TG,))


def _gate(xf, wg):
    vec = pl.BlockSpec((TG,), lambda s: (jnp.where(s < NT, s, s - NT),))
    return pl.pallas_call(
        _gate_kernel,
        grid=(2 * NT,),
        in_specs=[
            pl.BlockSpec((TG, D_), lambda s: (jnp.where(s < NT, s, 0), 0)),
            pl.BlockSpec((D_, E_), lambda s: (0, 0)),
        ],
        out_specs=[vec, vec, vec, vec,
                   pl.BlockSpec((3, SMAX), lambda s: (0, 0))],
        out_shape=[
            jax.ShapeDtypeStruct((T_,), jnp.float32),
            jax.ShapeDtypeStruct((T_,), jnp.float32),
            jax.ShapeDtypeStruct((T_,), jnp.int32),
            jax.ShapeDtypeStruct((T_,), jnp.int32),
            jax.ShapeDtypeStruct((3, SMAX), jnp.int32),
        ],
        scratch_shapes=[
            pltpu.VMEM((TG, TG), jnp.float32),
            pltpu.VMEM((E_, E_), jnp.float32),
            pltpu.VMEM((NT, TG, 1), jnp.int32),
            pltpu.VMEM((NT, TG, 1), jnp.int32),
            pltpu.VMEM((NT, TG, 1), jnp.float32),
            pltpu.VMEM((NT, TG, 1), jnp.float32),
            pltpu.VMEM((NT, TG, 1), jnp.float32),
            pltpu.VMEM((NT, TG, 1), jnp.float32),
            pltpu.VMEM((1, E_), jnp.float32),
        ],
    )(xf, wg)


def _gmm_kernel(m_ref, x_ref, w_ref, y_ref):
    s = pl.program_id(0)

    @pl.when(m_ref[2, s] == 1)
    def _():
        y_ref[...] = jnp.dot(x_ref[...].astype(jnp.bfloat16),
                             w_ref[0].astype(jnp.bfloat16),
                             preferred_element_type=jnp.float32)


def _gmm(meta, xs, we):
    grid_spec = pltpu.PrefetchScalarGridSpec(
        num_scalar_prefetch=1,
        grid=(SMAX,),
        in_specs=[
            pl.BlockSpec((TM, D_), lambda s, m: (m[0, s], 0)),
            pl.BlockSpec((1, D_, D_), lambda s, m: (m[1, s], 0, 0)),
        ],
        out_specs=pl.BlockSpec((TM, D_), lambda s, m: (m[0, s], 0)),
    )
    return pl.pallas_call(
        _gmm_kernel,
        grid_spec=grid_spec,
        out_shape=jax.ShapeDtypeStruct((NP_, D_), jnp.float32),
    )(meta, xs, we)


_NC, _NS = 2, 16
NW = _NC * _NS
TPW = T_ // NW
SB = 32
NSB = TPW // SB
_mesh = functools.partial(
    plsc.VectorSubcoreMesh, core_axis_name="c", subcore_axis_name="s")


def _dispatch(x2d, slot0, slot1):
    @functools.partial(
        pl.kernel,
        mesh=_mesh(),
        out_type=jax.ShapeDtypeStruct((NP_, D_), jnp.float32),
        scratch_types=[
            pltpu.VMEM((SB,), jnp.int32), pltpu.VMEM((SB,), jnp.int32),
            pltpu.VMEM((SB,), jnp.int32), pltpu.VMEM((SB,), jnp.int32),
            pltpu.VMEM((SB, D_), jnp.float32),
            pltpu.VMEM((SB, D_), jnp.float32),
            pltpu.SemaphoreType.DMA, pltpu.SemaphoreType.DMA,
            pltpu.SemaphoreType.DMA, pltpu.SemaphoreType.DMA,
        ],
    )
    def disp(x_hbm, s0_hbm, s1_hbm, xs_hbm,
             s0a, s1a, s0b, s1b, rows_a, rows_b, semA0, semA1, semB0, semB1):
        wid = lax.axis_index("s") * _NC + lax.axis_index("c")
        tb0 = wid * TPW
        s0v = (s0a, s0b)
        s1v = (s1a, s1b)
        rows = (rows_a, rows_b)
        sem0 = (semA0, semB0)
        sem1 = (semA1, semB1)

        def stage(b, st):
            tb = tb0 + b * SB
            @pl.when(b >= 2)
            def _():
                pltpu.make_async_copy(
                    rows[st], xs_hbm.at[s0v[st]], sem0[st]).wait()
                pltpu.make_async_copy(
                    rows[st], xs_hbm.at[s1v[st]], sem1[st]).wait()
            pltpu.sync_copy(s0_hbm.at[pl.ds(tb, SB)], s0v[st])
            pltpu.sync_copy(s1_hbm.at[pl.ds(tb, SB)], s1v[st])
            pltpu.sync_copy(x_hbm.at[pl.ds(tb, SB)], rows[st])
            pltpu.async_copy(rows[st], xs_hbm.at[s0v[st]], sem0[st])
            pltpu.async_copy(rows[st], xs_hbm.at[s1v[st]], sem1[st])

        def body(b2, carry):
            stage(b2 * 2, 0)
            stage(b2 * 2 + 1, 1)
            return carry

        lax.fori_loop(0, NSB // 2, body, 0)
        for st in range(2):
            pltpu.make_async_copy(rows[st], xs_hbm.at[s0v[st]], sem0[st]).wait()
            pltpu.make_async_copy(rows[st], xs_hbm.at[s1v[st]], sem1[st]).wait()

    return disp(x2d, slot0, slot1)


SB2 = 16
NSB2 = TPW // SB2


def _splat(vec16, lane16):
    dnums = lax.GatherDimensionNumbers(
        offset_dims=(), collapsed_slice_dims=(0,), start_index_map=(0,))
    return lax.gather(vec16, lane16[:, None], dnums, slice_sizes=(1,),
                      mode=lax.GatherScatterMode.PROMISE_IN_BOUNDS)


def _combine(ys, s0, s1, g0, g1):
    @functools.partial(
        pl.kernel,
        mesh=_mesh(),
        out_type=jax.ShapeDtypeStruct((T_, D_), jnp.float32),
        scratch_types=[
            pltpu.VMEM((SB2,), jnp.int32), pltpu.VMEM((SB2,), jnp.int32),
            pltpu.VMEM((SB2,), jnp.int32), pltpu.VMEM((SB2,), jnp.int32),
            pltpu.VMEM((SB2,), jnp.float32), pltpu.VMEM((SB2,), jnp.float32),
            pltpu.VMEM((SB2,), jnp.float32), pltpu.VMEM((SB2,), jnp.float32),
            pltpu.VMEM((SB2, D_), jnp.float32),
            pltpu.VMEM((SB2, D_), jnp.float32),
            pltpu.VMEM((SB2, D_), jnp.float32),
            pltpu.VMEM((SB2, D_), jnp.float32),
            pltpu.VMEM((SB2, D_), jnp.float32),
            pltpu.SemaphoreType.DMA, pltpu.SemaphoreType.DMA,
            pltpu.SemaphoreType.DMA, pltpu.SemaphoreType.DMA,
        ],
    )
    def comb(ys_hbm, s0_hbm, s1_hbm, g0_hbm, g1_hbm, out_hbm,
             s0a, s1a, s0b, s1b, g0a, g1a, g0b, g1b,
             r0a, r1a, r0b, r1b, o_v,
             semA0, semA1, semB0, semB1):
        wid = lax.axis_index("s") * _NC + lax.axis_index("c")
        tb0 = wid * TPW
        s0v = (s0a, s0b)
        s1v = (s1a, s1b)
        g0v = (g0a, g0b)
        g1v = (g1a, g1b)
        r0v = (r0a, r0b)
        r1v = (r1a, r1b)
        sem0 = (semA0, semB0)
        sem1 = (semA1, semB1)

        def issue(b, st):
            tb = tb0 + b * SB2
            pltpu.sync_copy(s0_hbm.at[pl.ds(tb, SB2)], s0v[st])
            pltpu.sync_copy(s1_hbm.at[pl.ds(tb, SB2)], s1v[st])
            pltpu.sync_copy(g0_hbm.at[pl.ds(tb, SB2)], g0v[st])
            pltpu.sync_copy(g1_hbm.at[pl.ds(tb, SB2)], g1v[st])
            pltpu.async_copy(ys_hbm.at[s0v[st]], r0v[st], sem0[st])
            pltpu.async_copy(ys_hbm.at[s1v[st]], r1v[st], sem1[st])

        def stage(b, st):
            @pl.when(b + 1 < NSB2)
            def _():
                issue(b + 1, st ^ 1)
            pltpu.make_async_copy(ys_hbm.at[s0v[st]], r0v[st], sem0[st]).wait()
            pltpu.make_async_copy(ys_hbm.at[s1v[st]], r1v[st], sem1[st]).wait()

            def row_body(i, carry):
                lane = jnp.full((16,), i, jnp.int32) & jnp.full((16,), 15, jnp.int32)
                ga = _splat(g0v[st][pl.ds(0, 16)], lane)
                gb = _splat(g1v[st][pl.ds(0, 16)], lane)
                for c in range(D_ // 16):
                    sl = pl.ds(c * 16, 16)
                    o_v[i, sl] = ga * r0v[st][i, sl] + gb * r1v[st][i, sl]
                return carry

            lax.fori_loop(0, SB2, row_body, 0)
            pltpu.sync_copy(o_v, out_hbm.at[pl.ds(tb0 + b * SB2, SB2)])

        issue(0, 0)

        def body(b2, carry):
            stage(b2 * 2, 0)
            stage(b2 * 2 + 1, 1)
            return carry

        lax.fori_loop(0, NSB2 // 2, body, 0)

    return comb(ys, s0, s1, g0, g1)


def kernel(x, Wg, We):
    xf = x.reshape(T_, D_)
    g0, g1, slot0, slot1, meta = _gate(xf, Wg)
    xs = _dispatch(xf, slot0, slot1)
    ys = _gmm(meta, xs, We)
    out = _combine(ys, slot0, slot1, g0, g1)
    return out.reshape(B_, L_, D_)


# speedup vs baseline: 1.1895x; 1.1895x over previous
import functools

import jax
import jax.numpy as jnp
from jax import lax
from jax.experimental import pallas as pl
from jax.experimental.pallas import tpu as pltpu
from jax.experimental.pallas import tpu_sc as plsc

B_, L_, D_ = 4, 2048, 1024
E_, K_ = 64, 2
T_ = B_ * L_
N_ = T_ * K_

TM = 512
SMAX = N_ // TM + E_
NP_ = SMAX * TM

TG = 512
NT = T_ // TG


def _gate_kernel(x_ref, wg_ref, g0_ref, g1_ref, s0_ref, s1_ref, meta_ref,
                 tril_ref, trilE_ref, i0s, i1s, r0s, r1s, g0s, g1s, cnt_ref):
    s = pl.program_id(0)
    b = jnp.where(s < NT, s, s - NT)

    @pl.when(s == 0)
    def _():
        row = lax.broadcasted_iota(jnp.int32, (TG, TG), 0)
        cc = lax.broadcasted_iota(jnp.int32, (TG, TG), 1)
        tril_ref[...] = (row > cc).astype(jnp.float32)
        er = lax.broadcasted_iota(jnp.int32, (E_, E_), 0)
        ec = lax.broadcasted_iota(jnp.int32, (E_, E_), 1)
        trilE_ref[...] = (er < ec).astype(jnp.float32)

    @pl.when(s < NT)
    def _():
        logits = jnp.dot(x_ref[...], wg_ref[...],
                         preferred_element_type=jnp.float32)
        sig = jax.nn.sigmoid(logits)
        col = lax.broadcasted_iota(jnp.int32, (TG, E_), 1)
        m1 = jnp.max(sig, axis=1, keepdims=True)
        i1 = jnp.min(jnp.where(sig == m1, col, E_), axis=1, keepdims=True)
        sig2 = jnp.where(col == i1, -1.0, sig)
        m2 = jnp.max(sig2, axis=1, keepdims=True)
        i2 = jnp.min(jnp.where(sig2 == m2, col, E_), axis=1, keepdims=True)
        o1 = (col == i1).astype(jnp.float32)
        o2 = (col == i2).astype(jnp.float32)
        o = o1 + o2
        cex = jnp.dot(tril_ref[...], o,
                      preferred_element_type=jnp.float32)
        prev = jnp.where(s == 0, 0.0, cnt_ref[...])
        r1 = jnp.sum((cex + prev) * o1, axis=1, keepdims=True)
        r2 = jnp.sum((cex + prev) * o2, axis=1, keepdims=True)
        i0s[b] = i1
        i1s[b] = i2
        r0s[b] = r1
        r1s[b] = r2
        g0s[b] = m1
        g1s[b] = m2
        cnt_ref[...] = prev + jnp.sum(o, axis=0, keepdims=True)

    @pl.when(s >= NT)
    def _():
        @pl.when(s == NT)
        def _():
            counts = cnt_ref[...]
            tcnt = jnp.floor((counts + (TM - 1)) * (1.0 / TM))
            base = jnp.dot(tcnt, trilE_ref[...],
                           preferred_element_type=jnp.float32)
            stot = jnp.sum(tcnt)
            s2 = lax.broadcasted_iota(jnp.int32, (SMAX, E_), 0).astype(jnp.float32)
            in_e = (s2 >= base) & (s2 < base + tcnt)
            eidsf = lax.broadcasted_iota(jnp.int32, (SMAX, E_), 1).astype(jnp.float32)
            gid = jnp.sum(jnp.where(in_e, eidsf, 0.0), axis=1)
            validv = jnp.sum(in_e.astype(jnp.float32), axis=1)
            sv1 = lax.broadcasted_iota(jnp.int32, (SMAX,), 0).astype(jnp.float32)
            lg = jnp.sum(jnp.where(sv1 == (stot - 1.0), gid, 0.0))
            meta_ref[0, :] = jnp.where(validv > 0, sv1, stot - 1.0).astype(jnp.int32)
            meta_ref[1, :] = jnp.where(validv > 0, gid, lg).astype(jnp.int32)
            meta_ref[2, :] = validv.astype(jnp.int32)
            cnt_ref[...] = base * TM

        col = lax.broadcasted_iota(jnp.int32, (TG, E_), 1)
        opad = cnt_ref[...]
        oh0 = (i0s[b] == col).astype(jnp.float32)
        oh1 = (i1s[b] == col).astype(jnp.float32)
        slot0 = jnp.sum(oh0 * opad, axis=1, keepdims=True) + r0s[b]
        slot1 = jnp.sum(oh1 * opad, axis=1, keepdims=True) + r1s[b]
        s0_ref[...] = jnp.reshape(slot0, (TG,)).astype(jnp.int32)
        s1_ref[...] = jnp.reshape(slot1, (TG,)).astype(jnp.int32)
        g0_ref[...] = jnp.reshape(g0s[b], (TG,))
        g1_ref[...] = jnp.reshape(g1s[b], (TG,))


def _gate(xf, wg):
    vec = pl.BlockSpec((TG,), lambda s: (jnp.where(s < NT, s, s - NT),))
    return pl.pallas_call(
        _gate_kernel,
        grid=(2 * NT,),
        in_specs=[
            pl.BlockSpec((TG, D_), lambda s: (jnp.where(s < NT, s, 0), 0)),
            pl.BlockSpec((D_, E_), lambda s: (0, 0)),
        ],
        out_specs=[vec, vec, vec, vec,
                   pl.BlockSpec((3, SMAX), lambda s: (0, 0))],
        out_shape=[
            jax.ShapeDtypeStruct((T_,), jnp.float32),
            jax.ShapeDtypeStruct((T_,), jnp.float32),
            jax.ShapeDtypeStruct((T_,), jnp.int32),
            jax.ShapeDtypeStruct((T_,), jnp.int32),
            jax.ShapeDtypeStruct((3, SMAX), jnp.int32),
        ],
        scratch_shapes=[
            pltpu.VMEM((TG, TG), jnp.float32),
            pltpu.VMEM((E_, E_), jnp.float32),
            pltpu.VMEM((NT, TG, 1), jnp.int32),
            pltpu.VMEM((NT, TG, 1), jnp.int32),
            pltpu.VMEM((NT, TG, 1), jnp.float32),
            pltpu.VMEM((NT, TG, 1), jnp.float32),
            pltpu.VMEM((NT, TG, 1), jnp.float32),
            pltpu.VMEM((NT, TG, 1), jnp.float32),
            pltpu.VMEM((1, E_), jnp.float32),
        ],
    )(xf, wg)


def _gmm_kernel(m_ref, x_ref, w_ref, y_ref):
    s = pl.program_id(0)

    @pl.when(m_ref[2, s] == 1)
    def _():
        y_ref[...] = jnp.dot(x_ref[...].astype(jnp.bfloat16),
                             w_ref[0].astype(jnp.bfloat16),
                             preferred_element_type=jnp.float32)


def _gmm(meta, xs, we):
    grid_spec = pltpu.PrefetchScalarGridSpec(
        num_scalar_prefetch=1,
        grid=(SMAX,),
        in_specs=[
            pl.BlockSpec((TM, D_), lambda s, m: (m[0, s], 0)),
            pl.BlockSpec((1, D_, D_), lambda s, m: (m[1, s], 0, 0)),
        ],
        out_specs=pl.BlockSpec((TM, D_), lambda s, m: (m[0, s], 0)),
    )
    return pl.pallas_call(
        _gmm_kernel,
        grid_spec=grid_spec,
        out_shape=jax.ShapeDtypeStruct((NP_, D_), jnp.float32),
    )(meta, xs, we)


_NC, _NS = 2, 16
NW = _NC * _NS
TPW = T_ // NW
SB = 32
NSB = TPW // SB
_mesh = functools.partial(
    plsc.VectorSubcoreMesh, core_axis_name="c", subcore_axis_name="s")


def _dispatch(x2d, slot0, slot1):
    @functools.partial(
        pl.kernel,
        mesh=_mesh(),
        out_type=jax.ShapeDtypeStruct((NP_, D_), jnp.float32),
        scratch_types=[
            pltpu.VMEM((SB,), jnp.int32), pltpu.VMEM((SB,), jnp.int32),
            pltpu.VMEM((SB,), jnp.int32), pltpu.VMEM((SB,), jnp.int32),
            pltpu.VMEM((SB, D_), jnp.float32),
            pltpu.VMEM((SB, D_), jnp.float32),
            pltpu.SemaphoreType.DMA, pltpu.SemaphoreType.DMA,
            pltpu.SemaphoreType.DMA, pltpu.SemaphoreType.DMA,
        ],
    )
    def disp(x_hbm, s0_hbm, s1_hbm, xs_hbm,
             s0a, s1a, s0b, s1b, rows_a, rows_b, semA0, semA1, semB0, semB1):
        wid = lax.axis_index("s") * _NC + lax.axis_index("c")
        tb0 = wid * TPW
        s0v = (s0a, s0b)
        s1v = (s1a, s1b)
        rows = (rows_a, rows_b)
        sem0 = (semA0, semB0)
        sem1 = (semA1, semB1)

        def stage(b, st):
            tb = tb0 + b * SB
            @pl.when(b >= 2)
            def _():
                pltpu.make_async_copy(
                    rows[st], xs_hbm.at[s0v[st]], sem0[st]).wait()
                pltpu.make_async_copy(
                    rows[st], xs_hbm.at[s1v[st]], sem1[st]).wait()
            pltpu.sync_copy(s0_hbm.at[pl.ds(tb, SB)], s0v[st])
            pltpu.sync_copy(s1_hbm.at[pl.ds(tb, SB)], s1v[st])
            pltpu.sync_copy(x_hbm.at[pl.ds(tb, SB)], rows[st])
            pltpu.async_copy(rows[st], xs_hbm.at[s0v[st]], sem0[st])
            pltpu.async_copy(rows[st], xs_hbm.at[s1v[st]], sem1[st])

        def body(b2, carry):
            stage(b2 * 2, 0)
            stage(b2 * 2 + 1, 1)
            return carry

        lax.fori_loop(0, NSB // 2, body, 0)
        for st in range(2):
            pltpu.make_async_copy(rows[st], xs_hbm.at[s0v[st]], sem0[st]).wait()
            pltpu.make_async_copy(rows[st], xs_hbm.at[s1v[st]], sem1[st]).wait()

    return disp(x2d, slot0, slot1)


SB2 = 16
NSB2 = TPW // SB2


def _splat(vec16, lane16):
    dnums = lax.GatherDimensionNumbers(
        offset_dims=(), collapsed_slice_dims=(0,), start_index_map=(0,))
    return lax.gather(vec16, lane16[:, None], dnums, slice_sizes=(1,),
                      mode=lax.GatherScatterMode.PROMISE_IN_BOUNDS)


def _combine(ys, s0, s1, g0, g1):
    @functools.partial(
        pl.kernel,
        mesh=_mesh(),
        out_type=jax.ShapeDtypeStruct((T_, D_), jnp.float32),
        scratch_types=[
            pltpu.VMEM((SB2,), jnp.int32), pltpu.VMEM((SB2,), jnp.int32),
            pltpu.VMEM((SB2,), jnp.int32), pltpu.VMEM((SB2,), jnp.int32),
            pltpu.VMEM((SB2,), jnp.float32), pltpu.VMEM((SB2,), jnp.float32),
            pltpu.VMEM((SB2,), jnp.float32), pltpu.VMEM((SB2,), jnp.float32),
            pltpu.VMEM((SB2, D_), jnp.float32),
            pltpu.VMEM((SB2, D_), jnp.float32),
            pltpu.VMEM((SB2, D_), jnp.float32),
            pltpu.VMEM((SB2, D_), jnp.float32),
            pltpu.VMEM((SB2, D_), jnp.float32),
            pltpu.SemaphoreType.DMA, pltpu.SemaphoreType.DMA,
            pltpu.SemaphoreType.DMA, pltpu.SemaphoreType.DMA,
        ],
    )
    def comb(ys_hbm, s0_hbm, s1_hbm, g0_hbm, g1_hbm, out_hbm,
             s0a, s1a, s0b, s1b, g0a, g1a, g0b, g1b,
             r0a, r1a, r0b, r1b, o_v,
             semA0, semA1, semB0, semB1):
        wid = lax.axis_index("s") * _NC + lax.axis_index("c")
        tb0 = wid * TPW
        s0v = (s0a, s0b)
        s1v = (s1a, s1b)
        g0v = (g0a, g0b)
        g1v = (g1a, g1b)
        r0v = (r0a, r0b)
        r1v = (r1a, r1b)
        sem0 = (semA0, semB0)
        sem1 = (semA1, semB1)

        def issue(b, st):
            tb = tb0 + b * SB2
            pltpu.sync_copy(s0_hbm.at[pl.ds(tb, SB2)], s0v[st])
            pltpu.sync_copy(s1_hbm.at[pl.ds(tb, SB2)], s1v[st])
            pltpu.sync_copy(g0_hbm.at[pl.ds(tb, SB2)], g0v[st])
            pltpu.sync_copy(g1_hbm.at[pl.ds(tb, SB2)], g1v[st])
            pltpu.async_copy(ys_hbm.at[s0v[st]], r0v[st], sem0[st])
            pltpu.async_copy(ys_hbm.at[s1v[st]], r1v[st], sem1[st])

        def stage(b, st):
            @pl.when(b + 1 < NSB2)
            def _():
                issue(b + 1, st ^ 1)
            pltpu.make_async_copy(ys_hbm.at[s0v[st]], r0v[st], sem0[st]).wait()
            pltpu.make_async_copy(ys_hbm.at[s1v[st]], r1v[st], sem1[st]).wait()

            def row_body(i, carry):
                lane = jnp.full((16,), i, jnp.int32) & jnp.full((16,), 15, jnp.int32)
                ga = _splat(g0v[st][pl.ds(0, 16)], lane)
                gb = _splat(g1v[st][pl.ds(0, 16)], lane)
                for c in range(D_ // 16):
                    sl = pl.ds(c * 16, 16)
                    o_v[i, sl] = ga * r0v[st][i, sl] + gb * r1v[st][i, sl]
                return carry

            lax.fori_loop(0, SB2, row_body, 0)
            pltpu.sync_copy(o_v, out_hbm.at[pl.ds(tb0 + b * SB2, SB2)])

        issue(0, 0)

        def body(b2, carry):
            stage(b2 * 2, 0)
            stage(b2 * 2 + 1, 1)
            return carry

        lax.fori_loop(0, NSB2 // 2, body, 0)

    return comb(ys, s0, s1, g0, g1)


def kernel(x, Wg, We):
    xf = x.reshape(T_, D_)
    g0, g1, slot0, slot1, meta = _gate(xf, Wg)
    xs = _dispatch(xf, slot0, slot1)
    ys = _gmm(meta, xs, We)
    out = _combine(ys, slot0, slot1, g0, g1)
    return out.reshape(B_, L_, D_)


# speedup vs baseline: 1.2701x; 1.0678x over previous
import functools

import jax
import jax.numpy as jnp
from jax import lax
from jax.experimental import pallas as pl
from jax.experimental.pallas import tpu as pltpu
from jax.experimental.pallas import tpu_sc as plsc

B_, L_, D_ = 4, 2048, 1024
E_, K_ = 64, 2
T_ = B_ * L_
N_ = T_ * K_

TM = 512
SMAX = N_ // TM + E_
NP_ = SMAX * TM

TG = 512
NT = T_ // TG


def _gate_kernel(x_ref, wg_ref, g0_ref, g1_ref, s0_ref, s1_ref, meta_ref,
                 tril_ref, trilE_ref, i0s, i1s, r0s, r1s, g0s, g1s, cnt_ref):
    s = pl.program_id(0)
    b = jnp.where(s < NT, s, s - NT)

    @pl.when(s == 0)
    def _():
        row = lax.broadcasted_iota(jnp.int32, (TG, TG), 0)
        cc = lax.broadcasted_iota(jnp.int32, (TG, TG), 1)
        tril_ref[...] = (row > cc).astype(jnp.float32)
        er = lax.broadcasted_iota(jnp.int32, (E_, E_), 0)
        ec = lax.broadcasted_iota(jnp.int32, (E_, E_), 1)
        trilE_ref[...] = (er < ec).astype(jnp.float32)

    @pl.when(s < NT)
    def _():
        logits = jnp.dot(x_ref[...], wg_ref[...],
                         preferred_element_type=jnp.float32)
        sig = jax.nn.sigmoid(logits)
        col = lax.broadcasted_iota(jnp.int32, (TG, E_), 1)
        m1 = jnp.max(sig, axis=1, keepdims=True)
        i1 = jnp.min(jnp.where(sig == m1, col, E_), axis=1, keepdims=True)
        sig2 = jnp.where(col == i1, -1.0, sig)
        m2 = jnp.max(sig2, axis=1, keepdims=True)
        i2 = jnp.min(jnp.where(sig2 == m2, col, E_), axis=1, keepdims=True)
        o1 = (col == i1).astype(jnp.float32)
        o2 = (col == i2).astype(jnp.float32)
        o = o1 + o2
        cex = jnp.dot(tril_ref[...], o,
                      preferred_element_type=jnp.float32)
        prev = jnp.where(s == 0, 0.0, cnt_ref[...])
        r1 = jnp.sum((cex + prev) * o1, axis=1, keepdims=True)
        r2 = jnp.sum((cex + prev) * o2, axis=1, keepdims=True)
        i0s[b] = i1
        i1s[b] = i2
        r0s[b] = r1
        r1s[b] = r2
        g0s[b] = m1
        g1s[b] = m2
        cnt_ref[...] = prev + jnp.sum(o, axis=0, keepdims=True)

    @pl.when(s >= NT)
    def _():
        @pl.when(s == NT)
        def _():
            counts = cnt_ref[...]
            tcnt = jnp.floor((counts + (TM - 1)) * (1.0 / TM))
            base = jnp.dot(tcnt, trilE_ref[...],
                           preferred_element_type=jnp.float32)
            stot = jnp.sum(tcnt)
            s2 = lax.broadcasted_iota(jnp.int32, (SMAX, E_), 0).astype(jnp.float32)
            in_e = (s2 >= base) & (s2 < base + tcnt)
            eidsf = lax.broadcasted_iota(jnp.int32, (SMAX, E_), 1).astype(jnp.float32)
            gid = jnp.sum(jnp.where(in_e, eidsf, 0.0), axis=1)
            validv = jnp.sum(in_e.astype(jnp.float32), axis=1)
            sv1 = lax.broadcasted_iota(jnp.int32, (SMAX,), 0).astype(jnp.float32)
            lg = jnp.sum(jnp.where(sv1 == (stot - 1.0), gid, 0.0))
            meta_ref[0, :] = jnp.where(validv > 0, sv1, stot - 1.0).astype(jnp.int32)
            meta_ref[1, :] = jnp.where(validv > 0, gid, lg).astype(jnp.int32)
            meta_ref[2, :] = validv.astype(jnp.int32)
            cnt_ref[...] = base * TM

        col = lax.broadcasted_iota(jnp.int32, (TG, E_), 1)
        opad = cnt_ref[...]
        oh0 = (i0s[b] == col).astype(jnp.float32)
        oh1 = (i1s[b] == col).astype(jnp.float32)
        slot0 = jnp.sum(oh0 * opad, axis=1, keepdims=True) + r0s[b]
        slot1 = jnp.sum(oh1 * opad, axis=1, keepdims=True) + r1s[b]
        s0_ref[...] = jnp.reshape(slot0, (TG,)).astype(jnp.int32)
        s1_ref[...] = jnp.reshape(slot1, (TG,)).astype(jnp.int32)
        g0_ref[...] = jnp.reshape(g0s[b], (TG,))
        g1_ref[...] = jnp.reshape(g1s[b], (TG,))


def _gate(xf, wg):
    vec = pl.BlockSpec((TG,), lambda s: (jnp.where(s < NT, s, s - NT),))
    return pl.pallas_call(
        _gate_kernel,
        grid=(2 * NT,),
        in_specs=[
            pl.BlockSpec((TG, D_), lambda s: (jnp.where(s < NT, s, 0), 0)),
            pl.BlockSpec((D_, E_), lambda s: (0, 0)),
        ],
        out_specs=[vec, vec, vec, vec,
                   pl.BlockSpec((3, SMAX), lambda s: (0, 0))],
        out_shape=[
            jax.ShapeDtypeStruct((T_,), jnp.float32),
            jax.ShapeDtypeStruct((T_,), jnp.float32),
            jax.ShapeDtypeStruct((T_,), jnp.int32),
            jax.ShapeDtypeStruct((T_,), jnp.int32),
            jax.ShapeDtypeStruct((3, SMAX), jnp.int32),
        ],
        scratch_shapes=[
            pltpu.VMEM((TG, TG), jnp.float32),
            pltpu.VMEM((E_, E_), jnp.float32),
            pltpu.VMEM((NT, TG, 1), jnp.int32),
            pltpu.VMEM((NT, TG, 1), jnp.int32),
            pltpu.VMEM((NT, TG, 1), jnp.float32),
            pltpu.VMEM((NT, TG, 1), jnp.float32),
            pltpu.VMEM((NT, TG, 1), jnp.float32),
            pltpu.VMEM((NT, TG, 1), jnp.float32),
            pltpu.VMEM((1, E_), jnp.float32),
        ],
    )(xf, wg)


def _gmm_kernel(m_ref, x_ref, w_ref, y_ref):
    s = pl.program_id(0)

    @pl.when(m_ref[2, s] == 1)
    def _():
        y_ref[...] = jnp.dot(x_ref[...].astype(jnp.bfloat16),
                             w_ref[0].astype(jnp.bfloat16),
                             preferred_element_type=jnp.float32)


def _gmm(meta, xs, we):
    grid_spec = pltpu.PrefetchScalarGridSpec(
        num_scalar_prefetch=1,
        grid=(SMAX,),
        in_specs=[
            pl.BlockSpec((TM, D_), lambda s, m: (m[0, s], 0)),
            pl.BlockSpec((1, D_, D_), lambda s, m: (m[1, s], 0, 0)),
        ],
        out_specs=pl.BlockSpec((TM, D_), lambda s, m: (m[0, s], 0)),
    )
    return pl.pallas_call(
        _gmm_kernel,
        grid_spec=grid_spec,
        out_shape=jax.ShapeDtypeStruct((NP_, D_), jnp.float32),
    )(meta, xs, we)


_NC, _NS = 2, 16
NW = _NC * _NS
TPW = T_ // NW
SB = 32
NSB = TPW // SB
_mesh = functools.partial(
    plsc.VectorSubcoreMesh, core_axis_name="c", subcore_axis_name="s")


def _dispatch(x2d, s0r, s1r):
    @functools.partial(
        pl.kernel,
        mesh=_mesh(),
        out_type=jax.ShapeDtypeStruct((NP_, D_), jnp.float32),
        scratch_types=[
            pltpu.VMEM((NSB, SB), jnp.int32), pltpu.VMEM((NSB, SB), jnp.int32),
            pltpu.VMEM((SB, D_), jnp.float32),
            pltpu.VMEM((SB, D_), jnp.float32),
            pltpu.SemaphoreType.DMA, pltpu.SemaphoreType.DMA,
            pltpu.SemaphoreType.DMA, pltpu.SemaphoreType.DMA,
        ],
    )
    def disp(x_hbm, s0_hbm, s1_hbm, xs_hbm,
             s0m, s1m, rows_a, rows_b, semA0, semA1, semB0, semB1):
        wid = lax.axis_index("s") * _NC + lax.axis_index("c")
        tb0 = wid * TPW
        rows = (rows_a, rows_b)
        sem0 = (semA0, semB0)
        sem1 = (semA1, semB1)
        pltpu.sync_copy(s0_hbm.at[wid], s0m)
        pltpu.sync_copy(s1_hbm.at[wid], s1m)

        def stage(b, st):
            @pl.when(b >= 2)
            def _():
                pltpu.make_async_copy(
                    rows[st], xs_hbm.at[s0m.at[b - 2]], sem0[st]).wait()
                pltpu.make_async_copy(
                    rows[st], xs_hbm.at[s1m.at[b - 2]], sem1[st]).wait()
            pltpu.sync_copy(x_hbm.at[pl.ds(tb0 + b * SB, SB)], rows[st])
            pltpu.async_copy(rows[st], xs_hbm.at[s0m.at[b]], sem0[st])
            pltpu.async_copy(rows[st], xs_hbm.at[s1m.at[b]], sem1[st])

        def body(b2, carry):
            stage(b2 * 2, 0)
            stage(b2 * 2 + 1, 1)
            return carry

        lax.fori_loop(0, NSB // 2, body, 0)
        for st in range(2):
            b = NSB - 2 + st
            pltpu.make_async_copy(rows[st], xs_hbm.at[s0m.at[b]], sem0[st]).wait()
            pltpu.make_async_copy(rows[st], xs_hbm.at[s1m.at[b]], sem1[st]).wait()

    return disp(x2d, s0r, s1r)


SB2 = 16
NSB2 = TPW // SB2


def _splat(vec16, lane16):
    dnums = lax.GatherDimensionNumbers(
        offset_dims=(), collapsed_slice_dims=(0,), start_index_map=(0,))
    return lax.gather(vec16, lane16[:, None], dnums, slice_sizes=(1,),
                      mode=lax.GatherScatterMode.PROMISE_IN_BOUNDS)


def _combine(ys, s0, s1, g0, g1):
    @functools.partial(
        pl.kernel,
        mesh=_mesh(),
        out_type=jax.ShapeDtypeStruct((T_, D_), jnp.float32),
        scratch_types=[
            pltpu.VMEM((TPW,), jnp.int32), pltpu.VMEM((TPW,), jnp.int32),
            pltpu.VMEM((TPW,), jnp.float32), pltpu.VMEM((TPW,), jnp.float32),
            pltpu.VMEM((SB2, D_), jnp.float32),
            pltpu.VMEM((SB2, D_), jnp.float32),
            pltpu.VMEM((SB2, D_), jnp.float32),
            pltpu.VMEM((SB2, D_), jnp.float32),
            pltpu.VMEM((SB2, D_), jnp.float32),
            pltpu.SemaphoreType.DMA, pltpu.SemaphoreType.DMA,
            pltpu.SemaphoreType.DMA, pltpu.SemaphoreType.DMA,
        ],
    )
    def comb(ys_hbm, s0_hbm, s1_hbm, g0_hbm, g1_hbm, out_hbm,
             s0w, s1w, g0w, g1w, r0a, r1a, r0b, r1b, o_v,
             semA0, semA1, semB0, semB1):
        wid = lax.axis_index("s") * _NC + lax.axis_index("c")
        tb0 = wid * TPW
        r0v = (r0a, r0b)
        r1v = (r1a, r1b)
        sem0 = (semA0, semB0)
        sem1 = (semA1, semB1)
        pltpu.sync_copy(s0_hbm.at[pl.ds(tb0, TPW)], s0w)
        pltpu.sync_copy(s1_hbm.at[pl.ds(tb0, TPW)], s1w)
        pltpu.sync_copy(g0_hbm.at[pl.ds(tb0, TPW)], g0w)
        pltpu.sync_copy(g1_hbm.at[pl.ds(tb0, TPW)], g1w)

        def issue(b, st):
            pltpu.async_copy(
                ys_hbm.at[s0w.at[pl.ds(b * SB2, SB2)]], r0v[st], sem0[st])
            pltpu.async_copy(
                ys_hbm.at[s1w.at[pl.ds(b * SB2, SB2)]], r1v[st], sem1[st])

        def stage(b, st):
            @pl.when(b + 1 < NSB2)
            def _():
                issue(b + 1, st ^ 1)
            pltpu.make_async_copy(
                ys_hbm.at[s0w.at[pl.ds(b * SB2, SB2)]], r0v[st], sem0[st]).wait()
            pltpu.make_async_copy(
                ys_hbm.at[s1w.at[pl.ds(b * SB2, SB2)]], r1v[st], sem1[st]).wait()

            def row_body(i, carry):
                lane = jnp.full((16,), i, jnp.int32) & jnp.full((16,), 15, jnp.int32)
                ga = _splat(g0w[pl.ds(b * SB2, 16)], lane)
                gb = _splat(g1w[pl.ds(b * SB2, 16)], lane)
                for c in range(D_ // 16):
                    sl = pl.ds(c * 16, 16)
                    o_v[i, sl] = ga * r0v[st][i, sl] + gb * r1v[st][i, sl]
                return carry

            lax.fori_loop(0, SB2, row_body, 0)
            pltpu.sync_copy(o_v, out_hbm.at[pl.ds(tb0 + b * SB2, SB2)])

        issue(0, 0)

        def body(b2, carry):
            stage(b2 * 2, 0)
            stage(b2 * 2 + 1, 1)
            return carry

        lax.fori_loop(0, NSB2 // 2, body, 0)

    return comb(ys, s0, s1, g0, g1)


def kernel(x, Wg, We):
    xf = x.reshape(T_, D_)
    g0, g1, slot0, slot1, meta = _gate(xf, Wg)
    xs = _dispatch(xf, slot0.reshape(NW, NSB, SB), slot1.reshape(NW, NSB, SB))
    ys = _gmm(meta, xs, We)
    out = _combine(ys, slot0, slot1, g0, g1)
    return out.reshape(B_, L_, D_)


# speedup vs baseline: 1.3566x; 1.0680x over previous
import functools

import jax
import jax.numpy as jnp
from jax import lax
from jax.experimental import pallas as pl
from jax.experimental.pallas import tpu as pltpu
from jax.experimental.pallas import tpu_sc as plsc

B_, L_, D_ = 4, 2048, 1024
E_, K_ = 64, 2
T_ = B_ * L_
N_ = T_ * K_

TM = 384
SMAX = N_ // TM + E_
NP_ = SMAX * TM

TG = 512
NT = T_ // TG


def _gate_kernel(x_ref, wg_ref, g0_ref, g1_ref, s0_ref, s1_ref, meta_ref,
                 tril_ref, trilE_ref, i0s, i1s, r0s, r1s, g0s, g1s, cnt_ref):
    s = pl.program_id(0)
    b = jnp.where(s < NT, s, s - NT)

    @pl.when(s == 0)
    def _():
        row = lax.broadcasted_iota(jnp.int32, (TG, TG), 0)
        cc = lax.broadcasted_iota(jnp.int32, (TG, TG), 1)
        tril_ref[...] = (row > cc).astype(jnp.float32)
        er = lax.broadcasted_iota(jnp.int32, (E_, E_), 0)
        ec = lax.broadcasted_iota(jnp.int32, (E_, E_), 1)
        trilE_ref[...] = (er < ec).astype(jnp.float32)

    @pl.when(s < NT)
    def _():
        logits = jnp.dot(x_ref[...], wg_ref[...],
                         preferred_element_type=jnp.float32)
        sig = jax.nn.sigmoid(logits)
        col = lax.broadcasted_iota(jnp.int32, (TG, E_), 1)
        m1 = jnp.max(sig, axis=1, keepdims=True)
        i1 = jnp.min(jnp.where(sig == m1, col, E_), axis=1, keepdims=True)
        sig2 = jnp.where(col == i1, -1.0, sig)
        m2 = jnp.max(sig2, axis=1, keepdims=True)
        i2 = jnp.min(jnp.where(sig2 == m2, col, E_), axis=1, keepdims=True)
        o1 = (col == i1).astype(jnp.float32)
        o2 = (col == i2).astype(jnp.float32)
        o = o1 + o2
        cex = jnp.dot(tril_ref[...], o,
                      preferred_element_type=jnp.float32)
        prev = jnp.where(s == 0, 0.0, cnt_ref[...])
        r1 = jnp.sum((cex + prev) * o1, axis=1, keepdims=True)
        r2 = jnp.sum((cex + prev) * o2, axis=1, keepdims=True)
        i0s[b] = i1
        i1s[b] = i2
        r0s[b] = r1
        r1s[b] = r2
        g0s[b] = m1
        g1s[b] = m2
        cnt_ref[...] = prev + jnp.sum(o, axis=0, keepdims=True)

    @pl.when(s >= NT)
    def _():
        @pl.when(s == NT)
        def _():
            counts = cnt_ref[...]
            tcnt = jnp.floor((counts + (TM - 1)) * (1.0 / TM))
            base = jnp.dot(tcnt, trilE_ref[...],
                           preferred_element_type=jnp.float32)
            stot = jnp.sum(tcnt)
            s2 = lax.broadcasted_iota(jnp.int32, (SMAX, E_), 0).astype(jnp.float32)
            in_e = (s2 >= base) & (s2 < base + tcnt)
            eidsf = lax.broadcasted_iota(jnp.int32, (SMAX, E_), 1).astype(jnp.float32)
            gid = jnp.sum(jnp.where(in_e, eidsf, 0.0), axis=1)
            validv = jnp.sum(in_e.astype(jnp.float32), axis=1)
            sv1 = lax.broadcasted_iota(jnp.int32, (SMAX,), 0).astype(jnp.float32)
            lg = jnp.sum(jnp.where(sv1 == (stot - 1.0), gid, 0.0))
            meta_ref[0, :] = jnp.where(validv > 0, sv1, stot - 1.0).astype(jnp.int32)
            meta_ref[1, :] = jnp.where(validv > 0, gid, lg).astype(jnp.int32)
            meta_ref[2, :] = validv.astype(jnp.int32)
            cnt_ref[...] = base * TM

        col = lax.broadcasted_iota(jnp.int32, (TG, E_), 1)
        opad = cnt_ref[...]
        oh0 = (i0s[b] == col).astype(jnp.float32)
        oh1 = (i1s[b] == col).astype(jnp.float32)
        slot0 = jnp.sum(oh0 * opad, axis=1, keepdims=True) + r0s[b]
        slot1 = jnp.sum(oh1 * opad, axis=1, keepdims=True) + r1s[b]
        s0_ref[...] = jnp.reshape(slot0, (TG,)).astype(jnp.int32)
        s1_ref[...] = jnp.reshape(slot1, (TG,)).astype(jnp.int32)
        g0_ref[...] = jnp.reshape(g0s[b], (TG,))
        g1_ref[...] = jnp.reshape(g1s[b], (TG,))


def _gate(xf, wg):
    vec = pl.BlockSpec((TG,), lambda s: (jnp.where(s < NT, s, s - NT),))
    return pl.pallas_call(
        _gate_kernel,
        grid=(2 * NT,),
        in_specs=[
            pl.BlockSpec((TG, D_), lambda s: (jnp.where(s < NT, s, 0), 0)),
            pl.BlockSpec((D_, E_), lambda s: (0, 0)),
        ],
        out_specs=[vec, vec, vec, vec,
                   pl.BlockSpec((3, SMAX), lambda s: (0, 0))],
        out_shape=[
            jax.ShapeDtypeStruct((T_,), jnp.float32),
            jax.ShapeDtypeStruct((T_,), jnp.float32),
            jax.ShapeDtypeStruct((T_,), jnp.int32),
            jax.ShapeDtypeStruct((T_,), jnp.int32),
            jax.ShapeDtypeStruct((3, SMAX), jnp.int32),
        ],
        scratch_shapes=[
            pltpu.VMEM((TG, TG), jnp.float32),
            pltpu.VMEM((E_, E_), jnp.float32),
            pltpu.VMEM((NT, TG, 1), jnp.int32),
            pltpu.VMEM((NT, TG, 1), jnp.int32),
            pltpu.VMEM((NT, TG, 1), jnp.float32),
            pltpu.VMEM((NT, TG, 1), jnp.float32),
            pltpu.VMEM((NT, TG, 1), jnp.float32),
            pltpu.VMEM((NT, TG, 1), jnp.float32),
            pltpu.VMEM((1, E_), jnp.float32),
        ],
    )(xf, wg)


def _gmm_kernel(m_ref, x_ref, w_ref, y_ref):
    s = pl.program_id(0)

    @pl.when(m_ref[2, s] == 1)
    def _():
        y_ref[...] = jnp.dot(x_ref[...].astype(jnp.bfloat16),
                             w_ref[0].astype(jnp.bfloat16),
                             preferred_element_type=jnp.float32)


def _gmm(meta, xs, we):
    grid_spec = pltpu.PrefetchScalarGridSpec(
        num_scalar_prefetch=1,
        grid=(SMAX,),
        in_specs=[
            pl.BlockSpec((TM, D_), lambda s, m: (m[0, s], 0)),
            pl.BlockSpec((1, D_, D_), lambda s, m: (m[1, s], 0, 0)),
        ],
        out_specs=pl.BlockSpec((TM, D_), lambda s, m: (m[0, s], 0)),
    )
    return pl.pallas_call(
        _gmm_kernel,
        grid_spec=grid_spec,
        out_shape=jax.ShapeDtypeStruct((NP_, D_), jnp.float32),
    )(meta, xs, we)


_NC, _NS = 2, 16
NW = _NC * _NS
TPW = T_ // NW
SB = 32
NSB = TPW // SB
_mesh = functools.partial(
    plsc.VectorSubcoreMesh, core_axis_name="c", subcore_axis_name="s")


def _dispatch(x2d, s0r, s1r):
    @functools.partial(
        pl.kernel,
        mesh=_mesh(),
        out_type=jax.ShapeDtypeStruct((NP_, D_), jnp.float32),
        scratch_types=[
            pltpu.VMEM((NSB, SB), jnp.int32), pltpu.VMEM((NSB, SB), jnp.int32),
            pltpu.VMEM((SB, D_), jnp.float32),
            pltpu.VMEM((SB, D_), jnp.float32),
            pltpu.SemaphoreType.DMA, pltpu.SemaphoreType.DMA,
            pltpu.SemaphoreType.DMA, pltpu.SemaphoreType.DMA,
        ],
    )
    def disp(x_hbm, s0_hbm, s1_hbm, xs_hbm,
             s0m, s1m, rows_a, rows_b, semA0, semA1, semB0, semB1):
        wid = lax.axis_index("s") * _NC + lax.axis_index("c")
        tb0 = wid * TPW
        rows = (rows_a, rows_b)
        sem0 = (semA0, semB0)
        sem1 = (semA1, semB1)
        pltpu.sync_copy(s0_hbm.at[wid], s0m)
        pltpu.sync_copy(s1_hbm.at[wid], s1m)

        def stage(b, st):
            @pl.when(b >= 2)
            def _():
                pltpu.make_async_copy(
                    rows[st], xs_hbm.at[s0m.at[b - 2]], sem0[st]).wait()
                pltpu.make_async_copy(
                    rows[st], xs_hbm.at[s1m.at[b - 2]], sem1[st]).wait()
            pltpu.sync_copy(x_hbm.at[pl.ds(tb0 + b * SB, SB)], rows[st])
            pltpu.async_copy(rows[st], xs_hbm.at[s0m.at[b]], sem0[st])
            pltpu.async_copy(rows[st], xs_hbm.at[s1m.at[b]], sem1[st])

        def body(b2, carry):
            stage(b2 * 2, 0)
            stage(b2 * 2 + 1, 1)
            return carry

        lax.fori_loop(0, NSB // 2, body, 0)
        for st in range(2):
            b = NSB - 2 + st
            pltpu.make_async_copy(rows[st], xs_hbm.at[s0m.at[b]], sem0[st]).wait()
            pltpu.make_async_copy(rows[st], xs_hbm.at[s1m.at[b]], sem1[st]).wait()

    return disp(x2d, s0r, s1r)


SB2 = 16
NSB2 = TPW // SB2


def _splat(vec16, lane16):
    dnums = lax.GatherDimensionNumbers(
        offset_dims=(), collapsed_slice_dims=(0,), start_index_map=(0,))
    return lax.gather(vec16, lane16[:, None], dnums, slice_sizes=(1,),
                      mode=lax.GatherScatterMode.PROMISE_IN_BOUNDS)


def _combine(ys, s0, s1, g0, g1):
    @functools.partial(
        pl.kernel,
        mesh=_mesh(),
        out_type=jax.ShapeDtypeStruct((T_, D_), jnp.float32),
        scratch_types=[
            pltpu.VMEM((TPW,), jnp.int32), pltpu.VMEM((TPW,), jnp.int32),
            pltpu.VMEM((TPW,), jnp.float32), pltpu.VMEM((TPW,), jnp.float32),
            pltpu.VMEM((SB2, D_), jnp.float32),
            pltpu.VMEM((SB2, D_), jnp.float32),
            pltpu.VMEM((SB2, D_), jnp.float32),
            pltpu.VMEM((SB2, D_), jnp.float32),
            pltpu.VMEM((SB2, D_), jnp.float32),
            pltpu.SemaphoreType.DMA, pltpu.SemaphoreType.DMA,
            pltpu.SemaphoreType.DMA, pltpu.SemaphoreType.DMA,
        ],
    )
    def comb(ys_hbm, s0_hbm, s1_hbm, g0_hbm, g1_hbm, out_hbm,
             s0w, s1w, g0w, g1w, r0a, r1a, r0b, r1b, o_v,
             semA0, semA1, semB0, semB1):
        wid = lax.axis_index("s") * _NC + lax.axis_index("c")
        tb0 = wid * TPW
        r0v = (r0a, r0b)
        r1v = (r1a, r1b)
        sem0 = (semA0, semB0)
        sem1 = (semA1, semB1)
        pltpu.sync_copy(s0_hbm.at[pl.ds(tb0, TPW)], s0w)
        pltpu.sync_copy(s1_hbm.at[pl.ds(tb0, TPW)], s1w)
        pltpu.sync_copy(g0_hbm.at[pl.ds(tb0, TPW)], g0w)
        pltpu.sync_copy(g1_hbm.at[pl.ds(tb0, TPW)], g1w)

        def issue(b, st):
            pltpu.async_copy(
                ys_hbm.at[s0w.at[pl.ds(b * SB2, SB2)]], r0v[st], sem0[st])
            pltpu.async_copy(
                ys_hbm.at[s1w.at[pl.ds(b * SB2, SB2)]], r1v[st], sem1[st])

        def stage(b, st):
            @pl.when(b + 1 < NSB2)
            def _():
                issue(b + 1, st ^ 1)
            pltpu.make_async_copy(
                ys_hbm.at[s0w.at[pl.ds(b * SB2, SB2)]], r0v[st], sem0[st]).wait()
            pltpu.make_async_copy(
                ys_hbm.at[s1w.at[pl.ds(b * SB2, SB2)]], r1v[st], sem1[st]).wait()

            def row_body(i, carry):
                lane = jnp.full((16,), i, jnp.int32) & jnp.full((16,), 15, jnp.int32)
                ga = _splat(g0w[pl.ds(b * SB2, 16)], lane)
                gb = _splat(g1w[pl.ds(b * SB2, 16)], lane)
                for c in range(D_ // 16):
                    sl = pl.ds(c * 16, 16)
                    o_v[i, sl] = ga * r0v[st][i, sl] + gb * r1v[st][i, sl]
                return carry

            lax.fori_loop(0, SB2, row_body, 0)
            pltpu.sync_copy(o_v, out_hbm.at[pl.ds(tb0 + b * SB2, SB2)])

        issue(0, 0)

        def body(b2, carry):
            stage(b2 * 2, 0)
            stage(b2 * 2 + 1, 1)
            return carry

        lax.fori_loop(0, NSB2 // 2, body, 0)

    return comb(ys, s0, s1, g0, g1)


def kernel(x, Wg, We):
    xf = x.reshape(T_, D_)
    g0, g1, slot0, slot1, meta = _gate(xf, Wg)
    xs = _dispatch(xf, slot0.reshape(NW, NSB, SB), slot1.reshape(NW, NSB, SB))
    ys = _gmm(meta, xs, We)
    out = _combine(ys, slot0, slot1, g0, g1)
    return out.reshape(B_, L_, D_)


# speedup vs baseline: 1.3865x; 1.0221x over previous
import functools

import jax
import jax.numpy as jnp
from jax import lax
from jax.experimental import pallas as pl
from jax.experimental.pallas import tpu as pltpu
from jax.experimental.pallas import tpu_sc as plsc

B_, L_, D_ = 4, 2048, 1024
E_, K_ = 64, 2
T_ = B_ * L_
N_ = T_ * K_

TM = 320
SMAX = N_ // TM + E_
NP_ = SMAX * TM

TG = 512
NT = T_ // TG


def _gate_kernel(x_ref, wg_ref, g0_ref, g1_ref, s0_ref, s1_ref, meta_ref,
                 tril_ref, trilE_ref, i0s, i1s, r0s, r1s, g0s, g1s, cnt_ref):
    s = pl.program_id(0)
    b = jnp.where(s < NT, s, s - NT)

    @pl.when(s == 0)
    def _():
        row = lax.broadcasted_iota(jnp.int32, (TG, TG), 0)
        cc = lax.broadcasted_iota(jnp.int32, (TG, TG), 1)
        tril_ref[...] = (row > cc).astype(jnp.float32)
        er = lax.broadcasted_iota(jnp.int32, (E_, E_), 0)
        ec = lax.broadcasted_iota(jnp.int32, (E_, E_), 1)
        trilE_ref[...] = (er < ec).astype(jnp.float32)

    @pl.when(s < NT)
    def _():
        logits = jnp.dot(x_ref[...], wg_ref[...],
                         preferred_element_type=jnp.float32)
        sig = jax.nn.sigmoid(logits)
        col = lax.broadcasted_iota(jnp.int32, (TG, E_), 1)
        m1 = jnp.max(sig, axis=1, keepdims=True)
        i1 = jnp.min(jnp.where(sig == m1, col, E_), axis=1, keepdims=True)
        sig2 = jnp.where(col == i1, -1.0, sig)
        m2 = jnp.max(sig2, axis=1, keepdims=True)
        i2 = jnp.min(jnp.where(sig2 == m2, col, E_), axis=1, keepdims=True)
        o1 = (col == i1).astype(jnp.float32)
        o2 = (col == i2).astype(jnp.float32)
        o = o1 + o2
        cex = jnp.dot(tril_ref[...], o,
                      preferred_element_type=jnp.float32)
        prev = jnp.where(s == 0, 0.0, cnt_ref[...])
        r1 = jnp.sum((cex + prev) * o1, axis=1, keepdims=True)
        r2 = jnp.sum((cex + prev) * o2, axis=1, keepdims=True)
        i0s[b] = i1
        i1s[b] = i2
        r0s[b] = r1
        r1s[b] = r2
        g0s[b] = m1
        g1s[b] = m2
        cnt_ref[...] = prev + jnp.sum(o, axis=0, keepdims=True)

    @pl.when(s >= NT)
    def _():
        @pl.when(s == NT)
        def _():
            counts = cnt_ref[...]
            tcnt = jnp.floor((counts + (TM - 1)) * (1.0 / TM))
            base = jnp.dot(tcnt, trilE_ref[...],
                           preferred_element_type=jnp.float32)
            stot = jnp.sum(tcnt)
            s2 = lax.broadcasted_iota(jnp.int32, (SMAX, E_), 0).astype(jnp.float32)
            in_e = (s2 >= base) & (s2 < base + tcnt)
            eidsf = lax.broadcasted_iota(jnp.int32, (SMAX, E_), 1).astype(jnp.float32)
            gid = jnp.sum(jnp.where(in_e, eidsf, 0.0), axis=1)
            validv = jnp.sum(in_e.astype(jnp.float32), axis=1)
            sv1 = lax.broadcasted_iota(jnp.int32, (SMAX,), 0).astype(jnp.float32)
            lg = jnp.sum(jnp.where(sv1 == (stot - 1.0), gid, 0.0))
            meta_ref[0, :] = jnp.where(validv > 0, sv1, stot - 1.0).astype(jnp.int32)
            meta_ref[1, :] = jnp.where(validv > 0, gid, lg).astype(jnp.int32)
            meta_ref[2, :] = validv.astype(jnp.int32)
            cnt_ref[...] = base * TM

        col = lax.broadcasted_iota(jnp.int32, (TG, E_), 1)
        opad = cnt_ref[...]
        oh0 = (i0s[b] == col).astype(jnp.float32)
        oh1 = (i1s[b] == col).astype(jnp.float32)
        slot0 = jnp.sum(oh0 * opad, axis=1, keepdims=True) + r0s[b]
        slot1 = jnp.sum(oh1 * opad, axis=1, keepdims=True) + r1s[b]
        s0_ref[...] = jnp.reshape(slot0, (TG,)).astype(jnp.int32)
        s1_ref[...] = jnp.reshape(slot1, (TG,)).astype(jnp.int32)
        g0_ref[...] = jnp.reshape(g0s[b], (TG,))
        g1_ref[...] = jnp.reshape(g1s[b], (TG,))


def _gate(xf, wg):
    vec = pl.BlockSpec((TG,), lambda s: (jnp.where(s < NT, s, s - NT),))
    return pl.pallas_call(
        _gate_kernel,
        grid=(2 * NT,),
        in_specs=[
            pl.BlockSpec((TG, D_), lambda s: (jnp.where(s < NT, s, 0), 0)),
            pl.BlockSpec((D_, E_), lambda s: (0, 0)),
        ],
        out_specs=[vec, vec, vec, vec,
                   pl.BlockSpec((3, SMAX), lambda s: (0, 0))],
        out_shape=[
            jax.ShapeDtypeStruct((T_,), jnp.float32),
            jax.ShapeDtypeStruct((T_,), jnp.float32),
            jax.ShapeDtypeStruct((T_,), jnp.int32),
            jax.ShapeDtypeStruct((T_,), jnp.int32),
            jax.ShapeDtypeStruct((3, SMAX), jnp.int32),
        ],
        scratch_shapes=[
            pltpu.VMEM((TG, TG), jnp.float32),
            pltpu.VMEM((E_, E_), jnp.float32),
            pltpu.VMEM((NT, TG, 1), jnp.int32),
            pltpu.VMEM((NT, TG, 1), jnp.int32),
            pltpu.VMEM((NT, TG, 1), jnp.float32),
            pltpu.VMEM((NT, TG, 1), jnp.float32),
            pltpu.VMEM((NT, TG, 1), jnp.float32),
            pltpu.VMEM((NT, TG, 1), jnp.float32),
            pltpu.VMEM((1, E_), jnp.float32),
        ],
    )(xf, wg)


def _gmm_kernel(m_ref, x_ref, w_ref, y_ref):
    s = pl.program_id(0)

    @pl.when(m_ref[2, s] == 1)
    def _():
        y_ref[...] = jnp.dot(x_ref[...].astype(jnp.bfloat16),
                             w_ref[0].astype(jnp.bfloat16),
                             preferred_element_type=jnp.float32)


def _gmm(meta, xs, we):
    grid_spec = pltpu.PrefetchScalarGridSpec(
        num_scalar_prefetch=1,
        grid=(SMAX,),
        in_specs=[
            pl.BlockSpec((TM, D_), lambda s, m: (m[0, s], 0)),
            pl.BlockSpec((1, D_, D_), lambda s, m: (m[1, s], 0, 0)),
        ],
        out_specs=pl.BlockSpec((TM, D_), lambda s, m: (m[0, s], 0)),
    )
    return pl.pallas_call(
        _gmm_kernel,
        grid_spec=grid_spec,
        out_shape=jax.ShapeDtypeStruct((NP_, D_), jnp.float32),
    )(meta, xs, we)


_NC, _NS = 2, 16
NW = _NC * _NS
TPW = T_ // NW
SB = 32
NSB = TPW // SB
_mesh = functools.partial(
    plsc.VectorSubcoreMesh, core_axis_name="c", subcore_axis_name="s")


def _dispatch(x2d, s0r, s1r):
    @functools.partial(
        pl.kernel,
        mesh=_mesh(),
        out_type=jax.ShapeDtypeStruct((NP_, D_), jnp.float32),
        scratch_types=[
            pltpu.VMEM((NSB, SB), jnp.int32), pltpu.VMEM((NSB, SB), jnp.int32),
            pltpu.VMEM((SB, D_), jnp.float32),
            pltpu.VMEM((SB, D_), jnp.float32),
            pltpu.SemaphoreType.DMA, pltpu.SemaphoreType.DMA,
            pltpu.SemaphoreType.DMA, pltpu.SemaphoreType.DMA,
        ],
    )
    def disp(x_hbm, s0_hbm, s1_hbm, xs_hbm,
             s0m, s1m, rows_a, rows_b, semA0, semA1, semB0, semB1):
        wid = lax.axis_index("s") * _NC + lax.axis_index("c")
        tb0 = wid * TPW
        rows = (rows_a, rows_b)
        sem0 = (semA0, semB0)
        sem1 = (semA1, semB1)
        pltpu.sync_copy(s0_hbm.at[wid], s0m)
        pltpu.sync_copy(s1_hbm.at[wid], s1m)

        def stage(b, st):
            @pl.when(b >= 2)
            def _():
                pltpu.make_async_copy(
                    rows[st], xs_hbm.at[s0m.at[b - 2]], sem0[st]).wait()
                pltpu.make_async_copy(
                    rows[st], xs_hbm.at[s1m.at[b - 2]], sem1[st]).wait()
            pltpu.sync_copy(x_hbm.at[pl.ds(tb0 + b * SB, SB)], rows[st])
            pltpu.async_copy(rows[st], xs_hbm.at[s0m.at[b]], sem0[st])
            pltpu.async_copy(rows[st], xs_hbm.at[s1m.at[b]], sem1[st])

        def body(b2, carry):
            stage(b2 * 2, 0)
            stage(b2 * 2 + 1, 1)
            return carry

        lax.fori_loop(0, NSB // 2, body, 0)
        for st in range(2):
            b = NSB - 2 + st
            pltpu.make_async_copy(rows[st], xs_hbm.at[s0m.at[b]], sem0[st]).wait()
            pltpu.make_async_copy(rows[st], xs_hbm.at[s1m.at[b]], sem1[st]).wait()

    return disp(x2d, s0r, s1r)


SB2 = 16
NSB2 = TPW // SB2


def _splat(vec16, lane16):
    dnums = lax.GatherDimensionNumbers(
        offset_dims=(), collapsed_slice_dims=(0,), start_index_map=(0,))
    return lax.gather(vec16, lane16[:, None], dnums, slice_sizes=(1,),
                      mode=lax.GatherScatterMode.PROMISE_IN_BOUNDS)


def _combine(ys, s0, s1, g0, g1):
    @functools.partial(
        pl.kernel,
        mesh=_mesh(),
        out_type=jax.ShapeDtypeStruct((T_, D_), jnp.float32),
        scratch_types=[
            pltpu.VMEM((TPW,), jnp.int32), pltpu.VMEM((TPW,), jnp.int32),
            pltpu.VMEM((TPW,), jnp.float32), pltpu.VMEM((TPW,), jnp.float32),
            pltpu.VMEM((SB2, D_), jnp.float32),
            pltpu.VMEM((SB2, D_), jnp.float32),
            pltpu.VMEM((SB2, D_), jnp.float32),
            pltpu.VMEM((SB2, D_), jnp.float32),
            pltpu.VMEM((SB2, D_), jnp.float32),
            pltpu.SemaphoreType.DMA, pltpu.SemaphoreType.DMA,
            pltpu.SemaphoreType.DMA, pltpu.SemaphoreType.DMA,
        ],
    )
    def comb(ys_hbm, s0_hbm, s1_hbm, g0_hbm, g1_hbm, out_hbm,
             s0w, s1w, g0w, g1w, r0a, r1a, r0b, r1b, o_v,
             semA0, semA1, semB0, semB1):
        wid = lax.axis_index("s") * _NC + lax.axis_index("c")
        tb0 = wid * TPW
        r0v = (r0a, r0b)
        r1v = (r1a, r1b)
        sem0 = (semA0, semB0)
        sem1 = (semA1, semB1)
        pltpu.sync_copy(s0_hbm.at[pl.ds(tb0, TPW)], s0w)
        pltpu.sync_copy(s1_hbm.at[pl.ds(tb0, TPW)], s1w)
        pltpu.sync_copy(g0_hbm.at[pl.ds(tb0, TPW)], g0w)
        pltpu.sync_copy(g1_hbm.at[pl.ds(tb0, TPW)], g1w)

        def issue(b, st):
            pltpu.async_copy(
                ys_hbm.at[s0w.at[pl.ds(b * SB2, SB2)]], r0v[st], sem0[st])
            pltpu.async_copy(
                ys_hbm.at[s1w.at[pl.ds(b * SB2, SB2)]], r1v[st], sem1[st])

        def stage(b, st):
            @pl.when(b + 1 < NSB2)
            def _():
                issue(b + 1, st ^ 1)
            pltpu.make_async_copy(
                ys_hbm.at[s0w.at[pl.ds(b * SB2, SB2)]], r0v[st], sem0[st]).wait()
            pltpu.make_async_copy(
                ys_hbm.at[s1w.at[pl.ds(b * SB2, SB2)]], r1v[st], sem1[st]).wait()

            def row_body(i, carry):
                lane = jnp.full((16,), i, jnp.int32) & jnp.full((16,), 15, jnp.int32)
                ga = _splat(g0w[pl.ds(b * SB2, 16)], lane)
                gb = _splat(g1w[pl.ds(b * SB2, 16)], lane)
                for c in range(D_ // 16):
                    sl = pl.ds(c * 16, 16)
                    o_v[i, sl] = ga * r0v[st][i, sl] + gb * r1v[st][i, sl]
                return carry

            lax.fori_loop(0, SB2, row_body, 0)
            pltpu.sync_copy(o_v, out_hbm.at[pl.ds(tb0 + b * SB2, SB2)])

        issue(0, 0)

        def body(b2, carry):
            stage(b2 * 2, 0)
            stage(b2 * 2 + 1, 1)
            return carry

        lax.fori_loop(0, NSB2 // 2, body, 0)

    return comb(ys, s0, s1, g0, g1)


def kernel(x, Wg, We):
    xf = x.reshape(T_, D_)
    g0, g1, slot0, slot1, meta = _gate(xf, Wg)
    xs = _dispatch(xf, slot0.reshape(NW, NSB, SB), slot1.reshape(NW, NSB, SB))
    ys = _gmm(meta, xs, We)
    out = _combine(ys, slot0, slot1, g0, g1)
    return out.reshape(B_, L_, D_)
